# baseline (speedup 1.0000x reference)
import functools

import jax
import jax.numpy as jnp
from jax.experimental import pallas as pl
from jax.experimental.pallas import tpu as pltpu
from jax.experimental.pallas import tpu_sc as plsc

VOCAB = 1000000
D_MODEL = 64
BATCH = 16384
HIST = 50
N_IDX = BATCH * HIST

ROWS_PER_STEP = 512
GRID_B = N_IDX // ROWS_PER_STEP

VBLK = 256
GRID_A = -(-VOCAB // VBLK)

_mesh = plsc.VectorSubcoreMesh(core_axis_name="core", subcore_axis_name="subcore")


@jax.jit
def _lookup(table, idx2):
    t2 = table.T

    @pl.kernel(
        out_type=jax.ShapeDtypeStruct((VOCAB // 2, 128), jnp.float32),
        mesh=_mesh,
        compiler_params=pltpu.CompilerParams(
            use_tc_tiling_on_sc=True, needs_layout_passes=False
        ),
    )
    def kern_a(t2_hbm, r_hbm):
        def body(tin, rout):
            def inner(tin2):
                iota = jax.lax.iota(jnp.int32, 16)

                @plsc.parallel_loop(0, D_MODEL, unroll=8)
                def _(d):
                    for k in range(VBLK // 16):
                        tin2[d, pl.ds(16 * k, 16)] = tin[d, pl.ds(16 * k, 16)]

                rvecs = [iota + 16 * k for k in range(4)]
                ones = jnp.full((16,), 1, jnp.int32)

                @plsc.parallel_loop(0, VBLK // 2, unroll=8)
                def _(q):
                    col0 = jnp.full((16,), 2 * q, jnp.int32)
                    cols = [col0, col0 + ones]
                    for half in range(2):
                        for k in range(4):
                            vals = plsc.load_gather(
                                tin2, [rvecs[k], cols[half]]
                            )
                            rout[q, pl.ds(64 * half + 16 * k, 16)] = vals

            pl.run_scoped(
                inner, pltpu.VMEM((D_MODEL, VBLK + 1), jnp.float32)
            )

        pltpu.emit_pipeline(
            body,
            grid=(GRID_A,),
            in_specs=[pl.BlockSpec((D_MODEL, VBLK), index_map=lambda j: (0, j))],
            out_specs=[
                pl.BlockSpec((VBLK // 2, 128), index_map=lambda j: (j, 0))
            ],
            core_axis_name=("core", "subcore"),
            dimension_semantics=(pltpu.PARALLEL,),
        )(t2_hbm, r_hbm)

    rows = kern_a(t2).reshape(VOCAB, D_MODEL)

    @pl.kernel(
        out_type=jax.ShapeDtypeStruct((N_IDX, D_MODEL), jnp.float32),
        mesh=_mesh,
        compiler_params=pltpu.CompilerParams(use_tc_tiling_on_sc=False),
    )
    def kern_b(table_hbm, i_hbm, o_hbm):
        def body(i_vmem, o_vmem):
            pltpu.sync_copy(table_hbm.at[i_vmem.at[0]], o_vmem)

        pltpu.emit_pipeline(
            body,
            grid=(GRID_B,),
            in_specs=[
                pl.BlockSpec((1, ROWS_PER_STEP), index_map=lambda i: (0, i))
            ],
            out_specs=[
                pl.BlockSpec((ROWS_PER_STEP, D_MODEL), index_map=lambda i: (i, 0))
            ],
            core_axis_name=("core", "subcore"),
            dimension_semantics=(pltpu.PARALLEL,),
        )(i_hbm, o_hbm)

    return kern_b(rows, idx2)


def kernel(x, table):
    out = _lookup(table, x.reshape(1, N_IDX))
    return out.reshape(BATCH, HIST, D_MODEL)


# speedup vs baseline: 1.2156x; 1.2156x over previous
import jax
import jax.numpy as jnp
from jax.experimental import pallas as pl
from jax.experimental.pallas import tpu as pltpu
from jax.experimental.pallas import tpu_sc as plsc

VOCAB = 1000000
D_MODEL = 64
BATCH = 16384
HIST = 50
N_IDX = BATCH * HIST

ROWS_PER_STEP = 640
GRID = N_IDX // ROWS_PER_STEP

_mesh = plsc.VectorSubcoreMesh(core_axis_name="core", subcore_axis_name="subcore")


@jax.jit
def _gather(table, idx2):
    @pl.kernel(
        out_type=jax.ShapeDtypeStruct((N_IDX, D_MODEL), jnp.float32),
        mesh=_mesh,
        compiler_params=pltpu.CompilerParams(use_tc_tiling_on_sc=False),
    )
    def kern(table_hbm, i_hbm, o_hbm):
        def body(i_vmem, o_vmem):
            pltpu.sync_copy(table_hbm.at[i_vmem.at[0]], o_vmem)

        pltpu.emit_pipeline(
            body,
            grid=(GRID,),
            in_specs=[
                pl.BlockSpec((1, ROWS_PER_STEP), index_map=lambda i: (0, i))
            ],
            out_specs=[
                pl.BlockSpec((ROWS_PER_STEP, D_MODEL), index_map=lambda i: (i, 0))
            ],
            core_axis_name=("core", "subcore"),
            dimension_semantics=(pltpu.PARALLEL,),
        )(i_hbm, o_hbm)

    return kern(table, idx2)


def kernel(x, table):
    out = _gather(table, x.reshape(1, N_IDX))
    return out.reshape(BATCH, HIST, D_MODEL)
